# Initial kernel scaffold; baseline (speedup 1.0000x reference)
import jax
import jax.numpy as jnp
from jax.experimental import pallas as pl


def kernel(x, edge_index, batch, W1, b1, W2, b2):
    raise NotImplementedError("write your pallas kernel here")


# speedup vs baseline: 12.9809x; 12.9809x over previous
import functools

import jax
import jax.numpy as jnp
from jax import lax
from jax.experimental import pallas as pl
from jax.experimental.pallas import tpu as pltpu
from jax.experimental.pallas import tpu_sc as plsc

N = 10000
D = 128
E = 320000
G = 64
NC = 2
NS = 16
NW = NC * NS
C = 128
CH = 80
EPW = CH * C
EPAD = NW * EPW
NPAD = 10240
RPT = NPAD // NS

def _zero_vec(ref, n):
    for k in range(n // 16):
        ref[pl.ds(k * 16, 16)] = jnp.zeros((16,), jnp.float32)


def _deg_body(dst_hbm, out_hbm, idx_v, ones_v, zer_v, acc_sh):
    cid = lax.axis_index("c")
    sid = lax.axis_index("s")
    wid = cid * NS + sid
    for k in range(C // 16):
        ones_v[pl.ds(k * 16, 16)] = jnp.ones((16,), jnp.float32)
    _zero_vec(zer_v, RPT)
    pltpu.sync_copy(zer_v, acc_sh.at[pl.ds(sid * RPT, RPT)])
    plsc.subcore_barrier()
    pltpu.sync_copy(dst_hbm.at[wid], idx_v)

    def chunk(j, carry):
        pltpu.sync_copy(ones_v, acc_sh.at[idx_v.at[j]], add=True)
        return carry

    lax.fori_loop(0, CH, chunk, 0)
    plsc.subcore_barrier()
    pltpu.sync_copy(acc_sh.at[pl.ds(sid * RPT, RPT)],
                    out_hbm.at[cid, pl.ds(sid * RPT, RPT)])


def _agg_body(g1_hbm, src_hbm, dst_hbm, out_hbm, sidx_v, didx_v, rows_v, acc_sh, sem):
    cid = lax.axis_index("c")
    sid = lax.axis_index("s")
    wid = cid * NS + sid

    def zrow(i, carry):
        for k in range(D // 16):
            rows_v[i, pl.ds(k * 16, 16)] = jnp.zeros((16,), jnp.float32)
        return carry

    lax.fori_loop(0, C, zrow, 0)
    for k in range(RPT // C):
        pltpu.sync_copy(rows_v, acc_sh.at[pl.ds(sid * RPT + k * C, C)])
    plsc.subcore_barrier()
    pltpu.sync_copy(src_hbm.at[wid], sidx_v)
    pltpu.sync_copy(dst_hbm.at[wid], didx_v)

    def chunk(j, carry):
        pltpu.async_copy(g1_hbm.at[sidx_v.at[j]], rows_v, sem).wait()
        pltpu.sync_copy(rows_v, acc_sh.at[didx_v.at[j]], add=True)
        return carry

    lax.fori_loop(0, CH, chunk, 0)
    plsc.subcore_barrier()
    for k in range(RPT // C):
        pltpu.sync_copy(acc_sh.at[pl.ds(sid * RPT + k * C, C)],
                        out_hbm.at[cid, pl.ds(sid * RPT + k * C, C)])


def _agg1_body(g2_hbm, src_hbm, dst_hbm, out_hbm, sidx_v, didx_v, vals_v, acc_sh, sem):
    cid = lax.axis_index("c")
    sid = lax.axis_index("s")
    wid = cid * NS + sid
    _zero_vec(vals_v, C)
    for k in range(RPT // C):
        pltpu.sync_copy(vals_v, acc_sh.at[pl.ds(sid * RPT + k * C, C)])
    plsc.subcore_barrier()
    pltpu.sync_copy(src_hbm.at[wid], sidx_v)
    pltpu.sync_copy(dst_hbm.at[wid], didx_v)

    def chunk(j, carry):
        pltpu.async_copy(g2_hbm.at[sidx_v.at[j]], vals_v, sem).wait()
        pltpu.sync_copy(vals_v, acc_sh.at[didx_v.at[j]], add=True)
        return carry

    lax.fori_loop(0, CH, chunk, 0)
    plsc.subcore_barrier()
    pltpu.sync_copy(acc_sh.at[pl.ds(sid * RPT, RPT)],
                    out_hbm.at[cid, pl.ds(sid * RPT, RPT)])


def _d1_body(x_ref, cnt_ref, w1_ref, h_ref, g_ref):
    h = jnp.dot(x_ref[...], w1_ref[...], preferred_element_type=jnp.float32)
    deg = cnt_ref[0, :] + cnt_ref[1, :] + 1.0
    dis = lax.rsqrt(deg)
    h_ref[...] = h
    g_ref[...] = h * dis[:, None]


_BLK = 256
_NB = NPAD // _BLK


def _dense1(x_pad, cnt, W1):
    return pl.pallas_call(
        _d1_body,
        grid=(_NB,),
        in_specs=[
            pl.BlockSpec((_BLK, D), lambda i: (i, 0)),
            pl.BlockSpec((NC, _BLK), lambda i: (0, i)),
            pl.BlockSpec((D, D), lambda i: (0, 0)),
        ],
        out_specs=[
            pl.BlockSpec((_BLK, D), lambda i: (i, 0)),
            pl.BlockSpec((_BLK, D), lambda i: (i, 0)),
        ],
        out_shape=[
            jax.ShapeDtypeStruct((NPAD, D), jnp.float32),
            jax.ShapeDtypeStruct((NPAD, D), jnp.float32),
        ],
    )(x_pad, cnt, W1)


def _d2_body(p_ref, h1_ref, cnt_ref, w2_ref, b1_ref, g2_ref, t_ref):
    deg = cnt_ref[0, :] + cnt_ref[1, :] + 1.0
    dis = lax.rsqrt(deg)
    inv = 1.0 / deg
    s = p_ref[0] + p_ref[1]
    out1 = s * dis[:, None] + h1_ref[...] * inv[:, None] + b1_ref[...][None, :]
    h = jnp.maximum(out1, 0.0)
    u = jnp.dot(h, w2_ref[...], preferred_element_type=jnp.float32)[:, 0]
    g2_ref[...] = u * dis
    t_ref[...] = u * inv


def _dense2(p, h1, cnt, W2, b1):
    return pl.pallas_call(
        _d2_body,
        grid=(_NB,),
        in_specs=[
            pl.BlockSpec((NC, _BLK, D), lambda i: (0, i, 0)),
            pl.BlockSpec((_BLK, D), lambda i: (i, 0)),
            pl.BlockSpec((NC, _BLK), lambda i: (0, i)),
            pl.BlockSpec((D, 1), lambda i: (0, 0)),
            pl.BlockSpec((D,), lambda i: (0,)),
        ],
        out_specs=[
            pl.BlockSpec((_BLK,), lambda i: (i,)),
            pl.BlockSpec((_BLK,), lambda i: (i,)),
        ],
        out_shape=[
            jax.ShapeDtypeStruct((NPAD,), jnp.float32),
            jax.ShapeDtypeStruct((NPAD,), jnp.float32),
        ],
    )(p, h1, cnt, W2, b1)


def _d3_body(q_ref, t_ref, cnt_ref, batch_ref, b2_ref, o_ref):
    deg = cnt_ref[0, :] + cnt_ref[1, :] + 1.0
    dis = lax.rsqrt(deg)
    out2 = dis * (q_ref[0] + q_ref[1]) + t_ref[...] + b2_ref[...]
    bt = batch_ref[...]
    oh = (bt[:, None] == lax.broadcasted_iota(jnp.int32, (NPAD, G), 1))
    ohf = oh.astype(jnp.float32)
    o_ref[...] = lax.dot_general(
        ohf, out2[:, None], (((0,), (0,)), ((), ())),
        preferred_element_type=jnp.float32)


def _dense3(q, t, cnt, batch_pad, b2):
    return pl.pallas_call(
        _d3_body,
        out_shape=jax.ShapeDtypeStruct((G, 1), jnp.float32),
    )(q, t, cnt, batch_pad, b2)


@functools.lru_cache(maxsize=1)
def _sc_kernels():
    mesh = plsc.VectorSubcoreMesh(
        core_axis_name="c", subcore_axis_name="s",
        num_cores=NC, num_subcores=NS)
    deg = pl.kernel(
        _deg_body,
        out_type=jax.ShapeDtypeStruct((NC, NPAD), jnp.float32),
        mesh=mesh,
        scratch_types=[
            pltpu.VMEM((CH, C), jnp.int32),
            pltpu.VMEM((C,), jnp.float32),
            pltpu.VMEM((RPT,), jnp.float32),
            pltpu.VMEM_SHARED((NPAD,), jnp.float32),
        ],
    )
    agg = pl.kernel(
        _agg_body,
        out_type=jax.ShapeDtypeStruct((NC, NPAD, D), jnp.float32),
        mesh=mesh,
        scratch_types=[
            pltpu.VMEM((CH, C), jnp.int32),
            pltpu.VMEM((CH, C), jnp.int32),
            pltpu.VMEM((C, D), jnp.float32),
            pltpu.VMEM_SHARED((NPAD, D), jnp.float32),
            pltpu.SemaphoreType.DMA,
        ],
    )
    agg1 = pl.kernel(
        _agg1_body,
        out_type=jax.ShapeDtypeStruct((NC, NPAD), jnp.float32),
        mesh=mesh,
        scratch_types=[
            pltpu.VMEM((CH, C), jnp.int32),
            pltpu.VMEM((CH, C), jnp.int32),
            pltpu.VMEM((C,), jnp.float32),
            pltpu.VMEM_SHARED((NPAD,), jnp.float32),
            pltpu.SemaphoreType.DMA,
        ],
    )
    return deg, agg, agg1


@jax.jit
def kernel(x, edge_index, batch, W1, b1, W2, b2):
    _deg, _agg, _agg1 = _sc_kernels()
    src = edge_index[0].astype(jnp.int32)
    dst = edge_index[1].astype(jnp.int32)
    pad_e = jnp.full((EPAD - E,), N, jnp.int32)
    srcm = jnp.concatenate([src, pad_e]).reshape(NW, CH, C)
    dstm = jnp.concatenate([dst, pad_e]).reshape(NW, CH, C)
    x_pad = jnp.concatenate([x, jnp.zeros((NPAD - N, D), jnp.float32)], axis=0)
    batch_pad = jnp.concatenate(
        [batch.astype(jnp.int32), jnp.full((NPAD - N,), G, jnp.int32)])

    cnt = _deg(dstm)
    h1, g1 = _dense1(x_pad, cnt, W1)
    p = _agg(g1, srcm, dstm)
    g2, t = _dense2(p, h1, cnt, W2, b1)
    q = _agg1(g2, srcm, dstm)
    return _dense3(q, t, cnt, batch_pad, b2)


# speedup vs baseline: 38.4663x; 2.9633x over previous
import functools

import jax
import jax.numpy as jnp
from jax import lax
from jax.experimental import pallas as pl
from jax.experimental.pallas import tpu as pltpu
from jax.experimental.pallas import tpu_sc as plsc

N = 10000
D = 128
E = 320000
G = 64
NC = 2
NS = 16
NW = NC * NS
C = 128
CH = 80
EPW = CH * C
EPAD = NW * EPW
NPAD = 10240
RPT = NPAD // NS

def _zero_vec(ref, n):
    for k in range(n // 16):
        ref[pl.ds(k * 16, 16)] = jnp.zeros((16,), jnp.float32)


def _deg_body(dst_hbm, out_hbm, idx_v, ones_v, zer_v, acc_sh):
    cid = lax.axis_index("c")
    sid = lax.axis_index("s")
    wid = cid * NS + sid
    for k in range(C // 16):
        ones_v[pl.ds(k * 16, 16)] = jnp.ones((16,), jnp.float32)
    _zero_vec(zer_v, RPT)
    pltpu.sync_copy(zer_v, acc_sh.at[pl.ds(sid * RPT, RPT)])
    plsc.subcore_barrier()
    pltpu.sync_copy(dst_hbm.at[wid], idx_v)

    def chunk(j, carry):
        pltpu.sync_copy(ones_v, acc_sh.at[idx_v.at[j]], add=True)
        return carry

    lax.fori_loop(0, CH, chunk, 0)
    plsc.subcore_barrier()
    pltpu.sync_copy(acc_sh.at[pl.ds(sid * RPT, RPT)],
                    out_hbm.at[cid, pl.ds(sid * RPT, RPT)])


GR = 8
NG = CH // GR


def _agg_body(g1_hbm, src_hbm, dst_hbm, out_hbm, sidx_v, didx_v, rows_a, rows_b,
              acc_sh, isem, sem_a, sem_b):
    cid = lax.axis_index("c")
    sid = lax.axis_index("s")
    wid = cid * NS + sid

    def zrow(i, carry):
        for k in range(D // 16):
            rows_a[i, pl.ds(k * 16, 16)] = jnp.zeros((16,), jnp.float32)
        return carry

    lax.fori_loop(0, C, zrow, 0)
    for k in range(RPT // C):
        pltpu.sync_copy(rows_a, acc_sh.at[pl.ds(sid * RPT + k * C, C)])
    plsc.subcore_barrier()

    pltpu.async_copy(src_hbm.at[wid, pl.ds(0, GR)], sidx_v.at[0], isem)
    pltpu.async_copy(dst_hbm.at[wid, pl.ds(0, GR)], didx_v.at[0], isem)

    def group(g, carry):
        b = lax.rem(g, 2)
        sg = sidx_v.at[b]
        dg = didx_v.at[b]
        pltpu.make_async_copy(src_hbm.at[wid, pl.ds(0, GR)], sg, isem).wait()
        pltpu.make_async_copy(dst_hbm.at[wid, pl.ds(0, GR)], dg, isem).wait()

        @pl.when(g < NG - 1)
        def _():
            pltpu.async_copy(
                src_hbm.at[wid, pl.ds((g + 1) * GR, GR)], sidx_v.at[1 - b], isem)
            pltpu.async_copy(
                dst_hbm.at[wid, pl.ds((g + 1) * GR, GR)], didx_v.at[1 - b], isem)

        pltpu.async_copy(g1_hbm.at[sg.at[0]], rows_a, sem_a)

        def pair(j, carry2):
            pltpu.async_copy(g1_hbm.at[sg.at[2 * j + 1]], rows_b, sem_b)
            pltpu.make_async_copy(g1_hbm.at[sg.at[2 * j]], rows_a, sem_a).wait()
            pltpu.sync_copy(rows_a, acc_sh.at[dg.at[2 * j]], add=True)

            @pl.when(j < GR // 2 - 1)
            def _():
                pltpu.async_copy(g1_hbm.at[sg.at[2 * j + 2]], rows_a, sem_a)

            pltpu.make_async_copy(g1_hbm.at[sg.at[2 * j + 1]], rows_b, sem_b).wait()
            pltpu.sync_copy(rows_b, acc_sh.at[dg.at[2 * j + 1]], add=True)
            return carry2

        lax.fori_loop(0, GR // 2, pair, 0)
        return carry

    lax.fori_loop(0, NG, group, 0)
    plsc.subcore_barrier()
    for k in range(RPT // C):
        pltpu.sync_copy(acc_sh.at[pl.ds(sid * RPT + k * C, C)],
                        out_hbm.at[cid, pl.ds(sid * RPT + k * C, C)])


def _agg1_body(g2_hbm, src_hbm, dst_hbm, out_hbm, sidx_v, didx_v, vals_a, vals_b,
               acc_sh, sem_a, sem_b):
    cid = lax.axis_index("c")
    sid = lax.axis_index("s")
    wid = cid * NS + sid
    _zero_vec(vals_a, C)
    for k in range(RPT // C):
        pltpu.sync_copy(vals_a, acc_sh.at[pl.ds(sid * RPT + k * C, C)])
    plsc.subcore_barrier()
    pltpu.sync_copy(src_hbm.at[wid], sidx_v)
    pltpu.sync_copy(dst_hbm.at[wid], didx_v)

    pltpu.async_copy(g2_hbm.at[sidx_v.at[0]], vals_a, sem_a)

    def pair(j, carry):
        pltpu.async_copy(g2_hbm.at[sidx_v.at[2 * j + 1]], vals_b, sem_b)
        pltpu.make_async_copy(g2_hbm.at[sidx_v.at[2 * j]], vals_a, sem_a).wait()
        pltpu.sync_copy(vals_a, acc_sh.at[didx_v.at[2 * j]], add=True)

        @pl.when(j < CH // 2 - 1)
        def _():
            pltpu.async_copy(g2_hbm.at[sidx_v.at[2 * j + 2]], vals_a, sem_a)

        pltpu.make_async_copy(g2_hbm.at[sidx_v.at[2 * j + 1]], vals_b, sem_b).wait()
        pltpu.sync_copy(vals_b, acc_sh.at[didx_v.at[2 * j + 1]], add=True)
        return carry

    lax.fori_loop(0, CH // 2, pair, 0)
    plsc.subcore_barrier()
    pltpu.sync_copy(acc_sh.at[pl.ds(sid * RPT, RPT)],
                    out_hbm.at[cid, pl.ds(sid * RPT, RPT)])


def _d1_body(x_ref, cnt_ref, w1_ref, h_ref, g_ref):
    h = jnp.dot(x_ref[...], w1_ref[...], preferred_element_type=jnp.float32)
    deg = cnt_ref[0, :] + cnt_ref[1, :] + 1.0
    dis = lax.rsqrt(deg)
    h_ref[...] = h
    g_ref[...] = h * dis[:, None]


_BLK = 256
_NB = NPAD // _BLK


def _dense1(x_pad, cnt, W1):
    return pl.pallas_call(
        _d1_body,
        grid=(_NB,),
        in_specs=[
            pl.BlockSpec((_BLK, D), lambda i: (i, 0)),
            pl.BlockSpec((NC, _BLK), lambda i: (0, i)),
            pl.BlockSpec((D, D), lambda i: (0, 0)),
        ],
        out_specs=[
            pl.BlockSpec((_BLK, D), lambda i: (i, 0)),
            pl.BlockSpec((_BLK, D), lambda i: (i, 0)),
        ],
        out_shape=[
            jax.ShapeDtypeStruct((NPAD, D), jnp.float32),
            jax.ShapeDtypeStruct((NPAD, D), jnp.float32),
        ],
    )(x_pad, cnt, W1)


def _d2_body(p_ref, h1_ref, cnt_ref, w2_ref, b1_ref, g2_ref, t_ref):
    deg = cnt_ref[0, :] + cnt_ref[1, :] + 1.0
    dis = lax.rsqrt(deg)
    inv = 1.0 / deg
    s = p_ref[0] + p_ref[1]
    out1 = s * dis[:, None] + h1_ref[...] * inv[:, None] + b1_ref[...][None, :]
    h = jnp.maximum(out1, 0.0)
    u = jnp.dot(h, w2_ref[...], preferred_element_type=jnp.float32)[:, 0]
    g2_ref[...] = u * dis
    t_ref[...] = u * inv


def _dense2(p, h1, cnt, W2, b1):
    return pl.pallas_call(
        _d2_body,
        grid=(_NB,),
        in_specs=[
            pl.BlockSpec((NC, _BLK, D), lambda i: (0, i, 0)),
            pl.BlockSpec((_BLK, D), lambda i: (i, 0)),
            pl.BlockSpec((NC, _BLK), lambda i: (0, i)),
            pl.BlockSpec((D, 1), lambda i: (0, 0)),
            pl.BlockSpec((D,), lambda i: (0,)),
        ],
        out_specs=[
            pl.BlockSpec((_BLK,), lambda i: (i,)),
            pl.BlockSpec((_BLK,), lambda i: (i,)),
        ],
        out_shape=[
            jax.ShapeDtypeStruct((NPAD,), jnp.float32),
            jax.ShapeDtypeStruct((NPAD,), jnp.float32),
        ],
    )(p, h1, cnt, W2, b1)


def _d3_body(q_ref, t_ref, cnt_ref, batch_ref, b2_ref, o_ref):
    deg = cnt_ref[0, :] + cnt_ref[1, :] + 1.0
    dis = lax.rsqrt(deg)
    out2 = dis * (q_ref[0] + q_ref[1]) + t_ref[...] + b2_ref[...]
    bt = batch_ref[...]
    oh = (bt[:, None] == lax.broadcasted_iota(jnp.int32, (NPAD, G), 1))
    ohf = oh.astype(jnp.float32)
    o_ref[...] = lax.dot_general(
        ohf, out2[:, None], (((0,), (0,)), ((), ())),
        preferred_element_type=jnp.float32)


def _dense3(q, t, cnt, batch_pad, b2):
    return pl.pallas_call(
        _d3_body,
        out_shape=jax.ShapeDtypeStruct((G, 1), jnp.float32),
    )(q, t, cnt, batch_pad, b2)


@functools.lru_cache(maxsize=1)
def _sc_kernels():
    mesh = plsc.VectorSubcoreMesh(
        core_axis_name="c", subcore_axis_name="s",
        num_cores=NC, num_subcores=NS)
    deg = pl.kernel(
        _deg_body,
        out_type=jax.ShapeDtypeStruct((NC, NPAD), jnp.float32),
        mesh=mesh,
        scratch_types=[
            pltpu.VMEM((CH, C), jnp.int32),
            pltpu.VMEM((C,), jnp.float32),
            pltpu.VMEM((RPT,), jnp.float32),
            pltpu.VMEM_SHARED((NPAD,), jnp.float32),
        ],
    )
    agg = pl.kernel(
        _agg_body,
        out_type=jax.ShapeDtypeStruct((NC, NPAD, D), jnp.float32),
        mesh=mesh,
        scratch_types=[
            pltpu.VMEM((2, GR, C), jnp.int32),
            pltpu.VMEM((2, GR, C), jnp.int32),
            pltpu.VMEM((C, D), jnp.float32),
            pltpu.VMEM((C, D), jnp.float32),
            pltpu.VMEM_SHARED((NPAD, D), jnp.float32),
            pltpu.SemaphoreType.DMA,
            pltpu.SemaphoreType.DMA,
            pltpu.SemaphoreType.DMA,
        ],
    )
    agg1 = pl.kernel(
        _agg1_body,
        out_type=jax.ShapeDtypeStruct((NC, NPAD), jnp.float32),
        mesh=mesh,
        scratch_types=[
            pltpu.VMEM((CH, C), jnp.int32),
            pltpu.VMEM((CH, C), jnp.int32),
            pltpu.VMEM((C,), jnp.float32),
            pltpu.VMEM((C,), jnp.float32),
            pltpu.VMEM_SHARED((NPAD,), jnp.float32),
            pltpu.SemaphoreType.DMA,
            pltpu.SemaphoreType.DMA,
        ],
    )
    return deg, agg, agg1


@jax.jit
def kernel(x, edge_index, batch, W1, b1, W2, b2):
    _deg, _agg, _agg1 = _sc_kernels()
    src = edge_index[0].astype(jnp.int32)
    dst = edge_index[1].astype(jnp.int32)
    pad_e = N + (jnp.arange(EPAD - E, dtype=jnp.int32) % (NPAD - N))
    srcm = jnp.concatenate([src, pad_e]).reshape(NW, CH, C)
    dstm = jnp.concatenate([dst, pad_e]).reshape(NW, CH, C)
    x_pad = jnp.concatenate([x, jnp.zeros((NPAD - N, D), jnp.float32)], axis=0)
    batch_pad = jnp.concatenate(
        [batch.astype(jnp.int32), jnp.full((NPAD - N,), G, jnp.int32)])

    cnt = _deg(dstm)
    h1, g1 = _dense1(x_pad, cnt, W1)
    p = _agg(g1, srcm, dstm)
    g2, t = _dense2(p, h1, cnt, W2, b1)
    q = _agg1(g2, srcm, dstm)
    return _dense3(q, t, cnt, batch_pad, b2)


# speedup vs baseline: 43.3119x; 1.1260x over previous
import functools

import jax
import jax.numpy as jnp
from jax import lax
from jax.experimental import pallas as pl
from jax.experimental.pallas import tpu as pltpu
from jax.experimental.pallas import tpu_sc as plsc

N = 10000
D = 128
E = 320000
G = 64
NC = 2
NS = 16
NW = NC * NS
C = 128
CH = 80
EPW = CH * C
EPAD = NW * EPW
NPAD = 10240
RPT = NPAD // NS

def _zero_vec(ref, n):
    for k in range(n // 16):
        ref[pl.ds(k * 16, 16)] = jnp.zeros((16,), jnp.float32)


def _deg_body(dst_hbm, out_hbm, idx_v, ones_v, zer_v, acc_sh):
    cid = lax.axis_index("c")
    sid = lax.axis_index("s")
    wid = cid * NS + sid
    for k in range(C // 16):
        ones_v[pl.ds(k * 16, 16)] = jnp.ones((16,), jnp.float32)
    _zero_vec(zer_v, RPT)
    pltpu.sync_copy(zer_v, acc_sh.at[pl.ds(sid * RPT, RPT)])
    plsc.subcore_barrier()
    pltpu.sync_copy(dst_hbm.at[wid], idx_v)

    def chunk(j, carry):
        pltpu.sync_copy(ones_v, acc_sh.at[idx_v.at[j]], add=True)
        return carry

    lax.fori_loop(0, CH, chunk, 0)
    plsc.subcore_barrier()
    pltpu.sync_copy(acc_sh.at[pl.ds(sid * RPT, RPT)],
                    out_hbm.at[cid, pl.ds(sid * RPT, RPT)])


GR = 8
NG = CH // GR


def _agg_body(g1_hbm, src_hbm, dst_hbm, p0_hbm, p1_hbm, sidx_v, didx_v, rows_a,
              rows_b, acc_sh, isem, sem_a, sem_b):
    cid = lax.axis_index("c")
    sid = lax.axis_index("s")
    wid = cid * NS + sid

    def zrow(i, carry):
        for k in range(D // 16):
            rows_a[i, pl.ds(k * 16, 16)] = jnp.zeros((16,), jnp.float32)
        return carry

    lax.fori_loop(0, C, zrow, 0)
    for k in range(RPT // C):
        pltpu.sync_copy(rows_a, acc_sh.at[pl.ds(sid * RPT + k * C, C)])
    plsc.subcore_barrier()

    pltpu.async_copy(src_hbm.at[wid, pl.ds(0, GR)], sidx_v.at[0], isem)
    pltpu.async_copy(dst_hbm.at[wid, pl.ds(0, GR)], didx_v.at[0], isem)

    def group(g, carry):
        b = lax.rem(g, 2)
        sg = sidx_v.at[b]
        dg = didx_v.at[b]
        pltpu.make_async_copy(src_hbm.at[wid, pl.ds(0, GR)], sg, isem).wait()
        pltpu.make_async_copy(dst_hbm.at[wid, pl.ds(0, GR)], dg, isem).wait()

        @pl.when(g < NG - 1)
        def _():
            pltpu.async_copy(
                src_hbm.at[wid, pl.ds((g + 1) * GR, GR)], sidx_v.at[1 - b], isem)
            pltpu.async_copy(
                dst_hbm.at[wid, pl.ds((g + 1) * GR, GR)], didx_v.at[1 - b], isem)

        pltpu.async_copy(g1_hbm.at[sg.at[0]], rows_a, sem_a)

        def pair(j, carry2):
            pltpu.async_copy(g1_hbm.at[sg.at[2 * j + 1]], rows_b, sem_b)
            pltpu.make_async_copy(g1_hbm.at[sg.at[2 * j]], rows_a, sem_a).wait()
            pltpu.sync_copy(rows_a, acc_sh.at[dg.at[2 * j]], add=True)

            @pl.when(j < GR // 2 - 1)
            def _():
                pltpu.async_copy(g1_hbm.at[sg.at[2 * j + 2]], rows_a, sem_a)

            pltpu.make_async_copy(g1_hbm.at[sg.at[2 * j + 1]], rows_b, sem_b).wait()
            pltpu.sync_copy(rows_b, acc_sh.at[dg.at[2 * j + 1]], add=True)
            return carry2

        lax.fori_loop(0, GR // 2, pair, 0)
        return carry

    lax.fori_loop(0, NG, group, 0)
    plsc.subcore_barrier()

    @pl.when(cid == 0)
    def _():
        pltpu.sync_copy(acc_sh.at[pl.ds(sid * RPT, RPT)],
                        p0_hbm.at[pl.ds(sid * RPT, RPT)])

    @pl.when(cid == 1)
    def _():
        pltpu.sync_copy(acc_sh.at[pl.ds(sid * RPT, RPT)],
                        p1_hbm.at[pl.ds(sid * RPT, RPT)])


def _agg1_body(g2_hbm, src_hbm, dst_hbm, q0_hbm, q1_hbm, sidx_v, didx_v, vals_a,
               vals_b, acc_sh, g2_sh, sem_a, sem_b):
    cid = lax.axis_index("c")
    sid = lax.axis_index("s")
    wid = cid * NS + sid
    _zero_vec(vals_a, C)
    for k in range(RPT // C):
        pltpu.sync_copy(vals_a, acc_sh.at[pl.ds(sid * RPT + k * C, C)])

    @pl.when(sid == 0)
    def _():
        pltpu.sync_copy(g2_hbm, g2_sh)

    plsc.subcore_barrier()
    pltpu.sync_copy(src_hbm.at[wid], sidx_v)
    pltpu.sync_copy(dst_hbm.at[wid], didx_v)

    pltpu.async_copy(g2_sh.at[sidx_v.at[0]], vals_a, sem_a)

    def pair(j, carry):
        pltpu.async_copy(g2_sh.at[sidx_v.at[2 * j + 1]], vals_b, sem_b)
        pltpu.make_async_copy(g2_sh.at[sidx_v.at[2 * j]], vals_a, sem_a).wait()
        pltpu.sync_copy(vals_a, acc_sh.at[didx_v.at[2 * j]], add=True)

        @pl.when(j < CH // 2 - 1)
        def _():
            pltpu.async_copy(g2_sh.at[sidx_v.at[2 * j + 2]], vals_a, sem_a)

        pltpu.make_async_copy(g2_sh.at[sidx_v.at[2 * j + 1]], vals_b, sem_b).wait()
        pltpu.sync_copy(vals_b, acc_sh.at[didx_v.at[2 * j + 1]], add=True)
        return carry

    lax.fori_loop(0, CH // 2, pair, 0)
    plsc.subcore_barrier()

    @pl.when(cid == 0)
    def _():
        pltpu.sync_copy(acc_sh.at[pl.ds(sid * RPT, RPT)],
                        q0_hbm.at[pl.ds(sid * RPT, RPT)])

    @pl.when(cid == 1)
    def _():
        pltpu.sync_copy(acc_sh.at[pl.ds(sid * RPT, RPT)],
                        q1_hbm.at[pl.ds(sid * RPT, RPT)])


def _d1_body(x_ref, cnt_ref, w1_ref, h_ref, g_ref):
    h = jnp.dot(x_ref[...], w1_ref[...], preferred_element_type=jnp.float32)
    deg = cnt_ref[0, :] + cnt_ref[1, :] + 1.0
    dis = lax.rsqrt(deg)
    h_ref[...] = h
    g_ref[...] = h * dis[:, None]


_BLK = 256
_NB = NPAD // _BLK


def _dense1(x_pad, cnt, W1):
    return pl.pallas_call(
        _d1_body,
        grid=(_NB,),
        in_specs=[
            pl.BlockSpec((_BLK, D), lambda i: (i, 0)),
            pl.BlockSpec((NC, _BLK), lambda i: (0, i)),
            pl.BlockSpec((D, D), lambda i: (0, 0)),
        ],
        out_specs=[
            pl.BlockSpec((_BLK, D), lambda i: (i, 0)),
            pl.BlockSpec((_BLK, D), lambda i: (i, 0)),
        ],
        out_shape=[
            jax.ShapeDtypeStruct((NPAD, D), jnp.float32),
            jax.ShapeDtypeStruct((NPAD, D), jnp.float32),
        ],
    )(x_pad, cnt, W1)


def _d2_body(p0_ref, p1_ref, h1_ref, cnt_ref, w2_ref, b1_ref, g2_ref, t_ref):
    deg = cnt_ref[0, :] + cnt_ref[1, :] + 1.0
    dis = lax.rsqrt(deg)
    inv = 1.0 / deg
    s = p0_ref[...] + p1_ref[...]
    out1 = s * dis[:, None] + h1_ref[...] * inv[:, None] + b1_ref[...][None, :]
    h = jnp.maximum(out1, 0.0)
    u = jnp.dot(h, w2_ref[...], preferred_element_type=jnp.float32)[:, 0]
    g2_ref[...] = u * dis
    t_ref[...] = u * inv


def _dense2(p0, p1, h1, cnt, W2, b1):
    return pl.pallas_call(
        _d2_body,
        grid=(_NB,),
        in_specs=[
            pl.BlockSpec((_BLK, D), lambda i: (i, 0)),
            pl.BlockSpec((_BLK, D), lambda i: (i, 0)),
            pl.BlockSpec((_BLK, D), lambda i: (i, 0)),
            pl.BlockSpec((NC, _BLK), lambda i: (0, i)),
            pl.BlockSpec((D, 1), lambda i: (0, 0)),
            pl.BlockSpec((D,), lambda i: (0,)),
        ],
        out_specs=[
            pl.BlockSpec((_BLK,), lambda i: (i,)),
            pl.BlockSpec((_BLK,), lambda i: (i,)),
        ],
        out_shape=[
            jax.ShapeDtypeStruct((NPAD,), jnp.float32),
            jax.ShapeDtypeStruct((NPAD,), jnp.float32),
        ],
    )(p0, p1, h1, cnt, W2, b1)


def _d3_body(q0_ref, q1_ref, t_ref, cnt_ref, batch_ref, b2_ref, o_ref):
    deg = cnt_ref[0, :] + cnt_ref[1, :] + 1.0
    dis = lax.rsqrt(deg)
    out2 = dis * (q0_ref[...] + q1_ref[...]) + t_ref[...] + b2_ref[...]
    bt = batch_ref[...]
    oh = (bt[:, None] == lax.broadcasted_iota(jnp.int32, (NPAD, G), 1))
    ohf = oh.astype(jnp.float32)
    o_ref[...] = lax.dot_general(
        ohf, out2[:, None], (((0,), (0,)), ((), ())),
        preferred_element_type=jnp.float32)


def _dense3(q0, q1, t, cnt, batch_pad, b2):
    return pl.pallas_call(
        _d3_body,
        out_shape=jax.ShapeDtypeStruct((G, 1), jnp.float32),
    )(q0, q1, t, cnt, batch_pad, b2)


@functools.lru_cache(maxsize=1)
def _sc_kernels():
    mesh = plsc.VectorSubcoreMesh(
        core_axis_name="c", subcore_axis_name="s",
        num_cores=NC, num_subcores=NS)
    deg = pl.kernel(
        _deg_body,
        out_type=jax.ShapeDtypeStruct((NC, NPAD), jnp.float32),
        mesh=mesh,
        scratch_types=[
            pltpu.VMEM((CH, C), jnp.int32),
            pltpu.VMEM((C,), jnp.float32),
            pltpu.VMEM((RPT,), jnp.float32),
            pltpu.VMEM_SHARED((NPAD,), jnp.float32),
        ],
    )
    agg = pl.kernel(
        _agg_body,
        out_type=(jax.ShapeDtypeStruct((NPAD, D), jnp.float32),
                  jax.ShapeDtypeStruct((NPAD, D), jnp.float32)),
        mesh=mesh,
        scratch_types=[
            pltpu.VMEM((2, GR, C), jnp.int32),
            pltpu.VMEM((2, GR, C), jnp.int32),
            pltpu.VMEM((C, D), jnp.float32),
            pltpu.VMEM((C, D), jnp.float32),
            pltpu.VMEM_SHARED((NPAD, D), jnp.float32),
            pltpu.SemaphoreType.DMA,
            pltpu.SemaphoreType.DMA,
            pltpu.SemaphoreType.DMA,
        ],
    )
    agg1 = pl.kernel(
        _agg1_body,
        out_type=(jax.ShapeDtypeStruct((NPAD,), jnp.float32),
                  jax.ShapeDtypeStruct((NPAD,), jnp.float32)),
        mesh=mesh,
        scratch_types=[
            pltpu.VMEM((CH, C), jnp.int32),
            pltpu.VMEM((CH, C), jnp.int32),
            pltpu.VMEM((C,), jnp.float32),
            pltpu.VMEM((C,), jnp.float32),
            pltpu.VMEM_SHARED((NPAD,), jnp.float32),
            pltpu.VMEM_SHARED((NPAD,), jnp.float32),
            pltpu.SemaphoreType.DMA,
            pltpu.SemaphoreType.DMA,
        ],
    )
    return deg, agg, agg1


@jax.jit
def kernel(x, edge_index, batch, W1, b1, W2, b2):
    _deg, _agg, _agg1 = _sc_kernels()
    src = edge_index[0].astype(jnp.int32)
    dst = edge_index[1].astype(jnp.int32)
    pad_e = N + (jnp.arange(EPAD - E, dtype=jnp.int32) % (NPAD - N))
    srcm = jnp.concatenate([src, pad_e]).reshape(NW, CH, C)
    dstm = jnp.concatenate([dst, pad_e]).reshape(NW, CH, C)
    x_pad = jnp.concatenate([x, jnp.zeros((NPAD - N, D), jnp.float32)], axis=0)
    batch_pad = jnp.concatenate(
        [batch.astype(jnp.int32), jnp.full((NPAD - N,), G, jnp.int32)])

    cnt = _deg(dstm)
    h1, g1 = _dense1(x_pad, cnt, W1)
    p0, p1 = _agg(g1, srcm, dstm)
    g2, t = _dense2(p0, p1, h1, cnt, W2, b1)
    q0, q1 = _agg1(g2, srcm, dstm)
    return _dense3(q0, q1, t, cnt, batch_pad, b2)


# speedup vs baseline: 49.3119x; 1.1385x over previous
import functools

import jax
import jax.numpy as jnp
from jax import lax
from jax.experimental import pallas as pl
from jax.experimental.pallas import tpu as pltpu
from jax.experimental.pallas import tpu_sc as plsc

N = 10000
D = 128
E = 320000
G = 64
NC = 2
NS = 16
NW = NC * NS
C = 128
CH = 82
EPW = CH * C
EPAD = NW * EPW
NPAD = 10240
RPT = NPAD // NS

def _zero_vec(ref, n):
    for k in range(n // 16):
        ref[pl.ds(k * 16, 16)] = jnp.zeros((16,), jnp.float32)


def _deg_body(dst_hbm, out_hbm, idx_v, ones_v, zer_v, acc_sh):
    cid = lax.axis_index("c")
    sid = lax.axis_index("s")
    wid = cid * NS + sid
    for k in range(C // 16):
        ones_v[pl.ds(k * 16, 16)] = jnp.ones((16,), jnp.float32)
    _zero_vec(zer_v, RPT)
    pltpu.sync_copy(zer_v, acc_sh.at[pl.ds(sid * RPT, RPT)])
    plsc.subcore_barrier()
    pltpu.sync_copy(dst_hbm.at[wid], idx_v)

    def chunk(j, carry):
        pltpu.sync_copy(ones_v, acc_sh.at[idx_v.at[j]], add=True)
        return carry

    lax.fori_loop(0, CH, chunk, 0)
    plsc.subcore_barrier()
    pltpu.sync_copy(acc_sh.at[pl.ds(sid * RPT, RPT)],
                    out_hbm.at[cid, pl.ds(sid * RPT, RPT)])


NP = CH // 2


def _agg_body(g1_hbm, src_hbm, dst_hbm, p0_hbm, p1_hbm, sidx_v, didx_v, rows_a,
              rows_b, acc_sh, isem, sem_a, sem_b):
    cid = lax.axis_index("c")
    sid = lax.axis_index("s")
    wid = cid * NS + sid

    def zrow(i, carry):
        for k in range(D // 16):
            rows_a[i, pl.ds(k * 16, 16)] = jnp.zeros((16,), jnp.float32)
        return carry

    lax.fori_loop(0, C, zrow, 0)
    for k in range(RPT // C):
        pltpu.sync_copy(rows_a, acc_sh.at[pl.ds(sid * RPT + k * C, C)])
    plsc.subcore_barrier()

    pltpu.sync_copy(src_hbm.at[wid, pl.ds(0, 2)], sidx_v.at[0])
    pltpu.sync_copy(dst_hbm.at[wid, pl.ds(0, 2)], didx_v.at[0])
    pltpu.async_copy(g1_hbm.at[sidx_v.at[0].at[0]], rows_a, sem_a)

    def pair(j, carry):
        b = lax.rem(j, 2)
        sg = sidx_v.at[b]
        dg = didx_v.at[b]

        @pl.when(j < NP - 1)
        def _():
            pltpu.async_copy(
                src_hbm.at[wid, pl.ds((j + 1) * 2, 2)], sidx_v.at[1 - b], isem)
            pltpu.async_copy(
                dst_hbm.at[wid, pl.ds((j + 1) * 2, 2)], didx_v.at[1 - b], isem)

        pltpu.async_copy(g1_hbm.at[sg.at[1]], rows_b, sem_b)
        pltpu.make_async_copy(g1_hbm.at[sg.at[0]], rows_a, sem_a).wait()
        pltpu.sync_copy(rows_a, acc_sh.at[dg.at[0]], add=True)

        @pl.when(j < NP - 1)
        def _():
            pltpu.make_async_copy(
                src_hbm.at[wid, pl.ds(0, 2)], sidx_v.at[1 - b], isem).wait()
            pltpu.make_async_copy(
                dst_hbm.at[wid, pl.ds(0, 2)], didx_v.at[1 - b], isem).wait()
            pltpu.async_copy(g1_hbm.at[sidx_v.at[1 - b].at[0]], rows_a, sem_a)

        pltpu.make_async_copy(g1_hbm.at[sg.at[1]], rows_b, sem_b).wait()
        pltpu.sync_copy(rows_b, acc_sh.at[dg.at[1]], add=True)
        return carry

    lax.fori_loop(0, NP, pair, 0)
    plsc.subcore_barrier()

    @pl.when(cid == 0)
    def _():
        pltpu.sync_copy(acc_sh.at[pl.ds(sid * RPT, RPT)],
                        p0_hbm.at[pl.ds(sid * RPT, RPT)])

    @pl.when(cid == 1)
    def _():
        pltpu.sync_copy(acc_sh.at[pl.ds(sid * RPT, RPT)],
                        p1_hbm.at[pl.ds(sid * RPT, RPT)])


def _agg1_body(g2_hbm, src_hbm, dst_hbm, q0_hbm, q1_hbm, sidx_v, didx_v, vals_a,
               vals_b, acc_sh, g2_sh, sem_a, sem_b):
    cid = lax.axis_index("c")
    sid = lax.axis_index("s")
    wid = cid * NS + sid
    _zero_vec(vals_a, C)
    for k in range(RPT // C):
        pltpu.sync_copy(vals_a, acc_sh.at[pl.ds(sid * RPT + k * C, C)])

    @pl.when(sid == 0)
    def _():
        pltpu.sync_copy(g2_hbm, g2_sh)

    plsc.subcore_barrier()
    pltpu.sync_copy(src_hbm.at[wid], sidx_v)
    pltpu.sync_copy(dst_hbm.at[wid], didx_v)

    pltpu.async_copy(g2_sh.at[sidx_v.at[0]], vals_a, sem_a)

    def pair(j, carry):
        pltpu.async_copy(g2_sh.at[sidx_v.at[2 * j + 1]], vals_b, sem_b)
        pltpu.make_async_copy(g2_sh.at[sidx_v.at[2 * j]], vals_a, sem_a).wait()
        pltpu.sync_copy(vals_a, acc_sh.at[didx_v.at[2 * j]], add=True)

        @pl.when(j < CH // 2 - 1)
        def _():
            pltpu.async_copy(g2_sh.at[sidx_v.at[2 * j + 2]], vals_a, sem_a)

        pltpu.make_async_copy(g2_sh.at[sidx_v.at[2 * j + 1]], vals_b, sem_b).wait()
        pltpu.sync_copy(vals_b, acc_sh.at[didx_v.at[2 * j + 1]], add=True)
        return carry

    lax.fori_loop(0, CH // 2, pair, 0)
    plsc.subcore_barrier()

    @pl.when(cid == 0)
    def _():
        pltpu.sync_copy(acc_sh.at[pl.ds(sid * RPT, RPT)],
                        q0_hbm.at[pl.ds(sid * RPT, RPT)])

    @pl.when(cid == 1)
    def _():
        pltpu.sync_copy(acc_sh.at[pl.ds(sid * RPT, RPT)],
                        q1_hbm.at[pl.ds(sid * RPT, RPT)])


def _d1_body(x_ref, cnt_ref, w1_ref, g_ref):
    h = jnp.dot(x_ref[...], w1_ref[...], preferred_element_type=jnp.float32)
    deg = cnt_ref[0, :] + cnt_ref[1, :]
    dis = lax.rsqrt(deg)
    g_ref[...] = h * dis[:, None]


_BLK = 512
_NB = NPAD // _BLK


def _dense1(x_pad, cnt, W1):
    return pl.pallas_call(
        _d1_body,
        grid=(_NB,),
        in_specs=[
            pl.BlockSpec((_BLK, D), lambda i: (i, 0)),
            pl.BlockSpec((NC, _BLK), lambda i: (0, i)),
            pl.BlockSpec((D, D), lambda i: (0, 0)),
        ],
        out_specs=pl.BlockSpec((_BLK, D), lambda i: (i, 0)),
        out_shape=jax.ShapeDtypeStruct((NPAD, D), jnp.float32),
    )(x_pad, cnt, W1)


def _d2_body(p0_ref, p1_ref, cnt_ref, w2_ref, b1_ref, g2_ref):
    deg = cnt_ref[0, :] + cnt_ref[1, :]
    dis = lax.rsqrt(deg)
    s = p0_ref[...] + p1_ref[...]
    out1 = s * dis[:, None] + b1_ref[...][None, :]
    h = jnp.maximum(out1, 0.0)
    u = jnp.dot(h, w2_ref[...], preferred_element_type=jnp.float32)[:, 0]
    g2_ref[...] = u * dis


def _dense2(p0, p1, cnt, W2, b1):
    return pl.pallas_call(
        _d2_body,
        grid=(_NB,),
        in_specs=[
            pl.BlockSpec((_BLK, D), lambda i: (i, 0)),
            pl.BlockSpec((_BLK, D), lambda i: (i, 0)),
            pl.BlockSpec((NC, _BLK), lambda i: (0, i)),
            pl.BlockSpec((D, 1), lambda i: (0, 0)),
            pl.BlockSpec((D,), lambda i: (0,)),
        ],
        out_specs=pl.BlockSpec((_BLK,), lambda i: (i,)),
        out_shape=jax.ShapeDtypeStruct((NPAD,), jnp.float32),
    )(p0, p1, cnt, W2, b1)


def _d3_body(q0_ref, q1_ref, cnt_ref, batch_ref, b2_ref, o_ref):
    i = pl.program_id(0)
    deg = cnt_ref[0, :] + cnt_ref[1, :]
    dis = lax.rsqrt(deg)
    out2 = dis * (q0_ref[...] + q1_ref[...]) + b2_ref[...]
    bt = batch_ref[...]
    oh = (bt[:, None] == lax.broadcasted_iota(jnp.int32, (_BLK, G), 1))
    ohf = oh.astype(jnp.float32)
    part = lax.dot_general(
        ohf, out2[:, None], (((0,), (0,)), ((), ())),
        preferred_element_type=jnp.float32)

    @pl.when(i == 0)
    def _():
        o_ref[...] = part

    @pl.when(i > 0)
    def _():
        o_ref[...] = o_ref[...] + part


def _dense3(q0, q1, cnt, batch_pad, b2):
    return pl.pallas_call(
        _d3_body,
        grid=(_NB,),
        in_specs=[
            pl.BlockSpec((_BLK,), lambda i: (i,)),
            pl.BlockSpec((_BLK,), lambda i: (i,)),
            pl.BlockSpec((NC, _BLK), lambda i: (0, i)),
            pl.BlockSpec((_BLK,), lambda i: (i,)),
            pl.BlockSpec((1,), lambda i: (0,)),
        ],
        out_specs=pl.BlockSpec((G, 1), lambda i: (0, 0)),
        out_shape=jax.ShapeDtypeStruct((G, 1), jnp.float32),
    )(q0, q1, cnt, batch_pad, b2)


@functools.lru_cache(maxsize=1)
def _sc_kernels():
    mesh = plsc.VectorSubcoreMesh(
        core_axis_name="c", subcore_axis_name="s",
        num_cores=NC, num_subcores=NS)
    deg = pl.kernel(
        _deg_body,
        out_type=jax.ShapeDtypeStruct((NC, NPAD), jnp.float32),
        mesh=mesh,
        scratch_types=[
            pltpu.VMEM((CH, C), jnp.int32),
            pltpu.VMEM((C,), jnp.float32),
            pltpu.VMEM((RPT,), jnp.float32),
            pltpu.VMEM_SHARED((NPAD,), jnp.float32),
        ],
    )
    agg = pl.kernel(
        _agg_body,
        out_type=(jax.ShapeDtypeStruct((NPAD, D), jnp.float32),
                  jax.ShapeDtypeStruct((NPAD, D), jnp.float32)),
        mesh=mesh,
        scratch_types=[
            pltpu.VMEM((2, 2, C), jnp.int32),
            pltpu.VMEM((2, 2, C), jnp.int32),
            pltpu.VMEM((C, D), jnp.float32),
            pltpu.VMEM((C, D), jnp.float32),
            pltpu.VMEM_SHARED((NPAD, D), jnp.float32),
            pltpu.SemaphoreType.DMA,
            pltpu.SemaphoreType.DMA,
            pltpu.SemaphoreType.DMA,
        ],
    )
    agg1 = pl.kernel(
        _agg1_body,
        out_type=(jax.ShapeDtypeStruct((NPAD,), jnp.float32),
                  jax.ShapeDtypeStruct((NPAD,), jnp.float32)),
        mesh=mesh,
        scratch_types=[
            pltpu.VMEM((CH, C), jnp.int32),
            pltpu.VMEM((CH, C), jnp.int32),
            pltpu.VMEM((C,), jnp.float32),
            pltpu.VMEM((C,), jnp.float32),
            pltpu.VMEM_SHARED((NPAD,), jnp.float32),
            pltpu.VMEM_SHARED((NPAD,), jnp.float32),
            pltpu.SemaphoreType.DMA,
            pltpu.SemaphoreType.DMA,
        ],
    )
    return deg, agg, agg1


@jax.jit
def kernel(x, edge_index, batch, W1, b1, W2, b2):
    _deg, _agg, _agg1 = _sc_kernels()
    src = edge_index[0].astype(jnp.int32)
    dst = edge_index[1].astype(jnp.int32)
    loop = jnp.arange(NPAD, dtype=jnp.int32)
    pad_e = N + (jnp.arange(EPAD - E - NPAD, dtype=jnp.int32) % (NPAD - N))
    srcm = jnp.concatenate([src, loop, pad_e]).reshape(NW, CH, C)
    dstm = jnp.concatenate([dst, loop, pad_e]).reshape(NW, CH, C)
    x_pad = jnp.concatenate([x, jnp.zeros((NPAD - N, D), jnp.float32)], axis=0)
    batch_pad = jnp.concatenate(
        [batch.astype(jnp.int32), jnp.full((NPAD - N,), G, jnp.int32)])

    cnt = _deg(dstm)
    g1 = _dense1(x_pad, cnt, W1)
    p0, p1 = _agg(g1, srcm, dstm)
    g2 = _dense2(p0, p1, cnt, W2, b1)
    q0, q1 = _agg1(g2, srcm, dstm)
    return _dense3(q0, q1, cnt, batch_pad, b2)


# speedup vs baseline: 52.9396x; 1.0736x over previous
import functools

import jax
import jax.numpy as jnp
from jax import lax
from jax.experimental import pallas as pl
from jax.experimental.pallas import tpu as pltpu
from jax.experimental.pallas import tpu_sc as plsc

N = 10000
D = 128
E = 320000
G = 64
NC = 2
NS = 16
NW = NC * NS
C = 128
CH = 82
EPW = CH * C
EPAD = NW * EPW
NPAD = 10240
RPT = NPAD // NS

def _zero_vec(ref, n):
    for k in range(n // 16):
        ref[pl.ds(k * 16, 16)] = jnp.zeros((16,), jnp.float32)


def _deg_body(dst_hbm, out_hbm, idx_v, ones_v, zer_v, acc_sh):
    cid = lax.axis_index("c")
    sid = lax.axis_index("s")
    wid = cid * NS + sid
    for k in range(C // 16):
        ones_v[pl.ds(k * 16, 16)] = jnp.ones((16,), jnp.float32)
    _zero_vec(zer_v, RPT)
    pltpu.sync_copy(zer_v, acc_sh.at[pl.ds(sid * RPT, RPT)])
    plsc.subcore_barrier()
    pltpu.sync_copy(dst_hbm.at[wid], idx_v)

    def chunk(j, carry):
        pltpu.sync_copy(ones_v, acc_sh.at[idx_v.at[j]], add=True)
        return carry

    lax.fori_loop(0, CH, chunk, 0)
    plsc.subcore_barrier()
    pltpu.sync_copy(acc_sh.at[pl.ds(sid * RPT, RPT)],
                    out_hbm.at[cid, pl.ds(sid * RPT, RPT)])


NP = CH // 2


def _agg_body(g1_hbm, src_hbm, dst_hbm, p0_hbm, p1_hbm, sidx_v, didx_v, rows_a,
              rows_b, acc_sh, isem, sem_a, sem_b):
    cid = lax.axis_index("c")
    sid = lax.axis_index("s")
    wid = cid * NS + sid

    def zrow(i, carry):
        for k in range(D // 16):
            rows_a[i, pl.ds(k * 16, 16)] = jnp.zeros((16,), jnp.float32)
        return carry

    lax.fori_loop(0, C, zrow, 0)
    for k in range(RPT // C):
        pltpu.sync_copy(rows_a, acc_sh.at[pl.ds(sid * RPT + k * C, C)])
    plsc.subcore_barrier()

    pltpu.sync_copy(src_hbm.at[wid, pl.ds(0, 2)], sidx_v.at[0])
    pltpu.sync_copy(dst_hbm.at[wid, pl.ds(0, 2)], didx_v.at[0])
    pltpu.async_copy(g1_hbm.at[sidx_v.at[0].at[0]], rows_a, sem_a)

    def pair(j, carry):
        b = lax.rem(j, 2)
        sg = sidx_v.at[b]
        dg = didx_v.at[b]

        @pl.when(j < NP - 1)
        def _():
            pltpu.async_copy(
                src_hbm.at[wid, pl.ds((j + 1) * 2, 2)], sidx_v.at[1 - b], isem)
            pltpu.async_copy(
                dst_hbm.at[wid, pl.ds((j + 1) * 2, 2)], didx_v.at[1 - b], isem)

        pltpu.async_copy(g1_hbm.at[sg.at[1]], rows_b, sem_b)
        pltpu.make_async_copy(g1_hbm.at[sg.at[0]], rows_a, sem_a).wait()
        pltpu.sync_copy(rows_a, acc_sh.at[dg.at[0]], add=True)

        @pl.when(j < NP - 1)
        def _():
            pltpu.make_async_copy(
                src_hbm.at[wid, pl.ds(0, 2)], sidx_v.at[1 - b], isem).wait()
            pltpu.make_async_copy(
                dst_hbm.at[wid, pl.ds(0, 2)], didx_v.at[1 - b], isem).wait()
            pltpu.async_copy(g1_hbm.at[sidx_v.at[1 - b].at[0]], rows_a, sem_a)

        pltpu.make_async_copy(g1_hbm.at[sg.at[1]], rows_b, sem_b).wait()
        pltpu.sync_copy(rows_b, acc_sh.at[dg.at[1]], add=True)
        return carry

    lax.fori_loop(0, NP, pair, 0)
    plsc.subcore_barrier()

    @pl.when(cid == 0)
    def _():
        pltpu.sync_copy(acc_sh.at[pl.ds(sid * RPT, RPT)],
                        p0_hbm.at[pl.ds(sid * RPT, RPT)])

    @pl.when(cid == 1)
    def _():
        pltpu.sync_copy(acc_sh.at[pl.ds(sid * RPT, RPT)],
                        p1_hbm.at[pl.ds(sid * RPT, RPT)])


def _agg1_body(g2_hbm, src_hbm, dst_hbm, q0_hbm, q1_hbm, sidx_v, didx_v, vals_a,
               vals_b, acc_sh, g2_sh, sem_a, sem_b):
    cid = lax.axis_index("c")
    sid = lax.axis_index("s")
    wid = cid * NS + sid
    _zero_vec(vals_a, C)
    for k in range(RPT // C):
        pltpu.sync_copy(vals_a, acc_sh.at[pl.ds(sid * RPT + k * C, C)])

    @pl.when(sid == 0)
    def _():
        pltpu.sync_copy(g2_hbm, g2_sh)

    plsc.subcore_barrier()
    pltpu.sync_copy(src_hbm.at[wid], sidx_v)
    pltpu.sync_copy(dst_hbm.at[wid], didx_v)

    pltpu.async_copy(g2_sh.at[sidx_v.at[0]], vals_a, sem_a)

    def pair(j, carry):
        pltpu.async_copy(g2_sh.at[sidx_v.at[2 * j + 1]], vals_b, sem_b)
        pltpu.make_async_copy(g2_sh.at[sidx_v.at[2 * j]], vals_a, sem_a).wait()
        pltpu.sync_copy(vals_a, acc_sh.at[didx_v.at[2 * j]], add=True)

        @pl.when(j < CH // 2 - 1)
        def _():
            pltpu.async_copy(g2_sh.at[sidx_v.at[2 * j + 2]], vals_a, sem_a)

        pltpu.make_async_copy(g2_sh.at[sidx_v.at[2 * j + 1]], vals_b, sem_b).wait()
        pltpu.sync_copy(vals_b, acc_sh.at[didx_v.at[2 * j + 1]], add=True)
        return carry

    lax.fori_loop(0, CH // 2, pair, 0)
    plsc.subcore_barrier()

    @pl.when(cid == 0)
    def _():
        pltpu.sync_copy(acc_sh.at[pl.ds(sid * RPT, RPT)],
                        q0_hbm.at[pl.ds(sid * RPT, RPT)])

    @pl.when(cid == 1)
    def _():
        pltpu.sync_copy(acc_sh.at[pl.ds(sid * RPT, RPT)],
                        q1_hbm.at[pl.ds(sid * RPT, RPT)])


def _d1_body(x_ref, cnt_ref, w1_ref, g_ref):
    h = jnp.dot(x_ref[...], w1_ref[...], preferred_element_type=jnp.float32)
    deg = cnt_ref[0, :] + cnt_ref[1, :]
    dis = lax.rsqrt(deg)
    g_ref[...] = h * dis[:, None]


_BLK = 1024
_NB = NPAD // _BLK


def _dense1(x_pad, cnt, W1):
    return pl.pallas_call(
        _d1_body,
        grid=(_NB,),
        in_specs=[
            pl.BlockSpec((_BLK, D), lambda i: (i, 0)),
            pl.BlockSpec((NC, _BLK), lambda i: (0, i)),
            pl.BlockSpec((D, D), lambda i: (0, 0)),
        ],
        out_specs=pl.BlockSpec((_BLK, D), lambda i: (i, 0)),
        out_shape=jax.ShapeDtypeStruct((NPAD, D), jnp.float32),
    )(x_pad, cnt, W1)


def _d2_body(p0_ref, p1_ref, cnt_ref, w2_ref, b1_ref, g2_ref):
    deg = cnt_ref[0, :] + cnt_ref[1, :]
    dis = lax.rsqrt(deg)
    s = p0_ref[...] + p1_ref[...]
    out1 = s * dis[:, None] + b1_ref[...][None, :]
    h = jnp.maximum(out1, 0.0)
    u = jnp.dot(h, w2_ref[...], preferred_element_type=jnp.float32)[:, 0]
    g2_ref[...] = u * dis


def _dense2(p0, p1, cnt, W2, b1):
    return pl.pallas_call(
        _d2_body,
        grid=(_NB,),
        in_specs=[
            pl.BlockSpec((_BLK, D), lambda i: (i, 0)),
            pl.BlockSpec((_BLK, D), lambda i: (i, 0)),
            pl.BlockSpec((NC, _BLK), lambda i: (0, i)),
            pl.BlockSpec((D, 1), lambda i: (0, 0)),
            pl.BlockSpec((D,), lambda i: (0,)),
        ],
        out_specs=pl.BlockSpec((_BLK,), lambda i: (i,)),
        out_shape=jax.ShapeDtypeStruct((NPAD,), jnp.float32),
    )(p0, p1, cnt, W2, b1)


def _d3_body(q0_ref, q1_ref, cnt_ref, batch_ref, b2_ref, o_ref):
    i = pl.program_id(0)
    deg = cnt_ref[0, :] + cnt_ref[1, :]
    dis = lax.rsqrt(deg)
    out2 = dis * (q0_ref[...] + q1_ref[...]) + b2_ref[...]
    bt = batch_ref[...]
    oh = (bt[:, None] == lax.broadcasted_iota(jnp.int32, (_BLK, G), 1))
    ohf = oh.astype(jnp.float32)
    part = lax.dot_general(
        ohf, out2[:, None], (((0,), (0,)), ((), ())),
        preferred_element_type=jnp.float32)

    @pl.when(i == 0)
    def _():
        o_ref[...] = part

    @pl.when(i > 0)
    def _():
        o_ref[...] = o_ref[...] + part


def _dense3(q0, q1, cnt, batch_pad, b2):
    return pl.pallas_call(
        _d3_body,
        grid=(_NB,),
        in_specs=[
            pl.BlockSpec((_BLK,), lambda i: (i,)),
            pl.BlockSpec((_BLK,), lambda i: (i,)),
            pl.BlockSpec((NC, _BLK), lambda i: (0, i)),
            pl.BlockSpec((_BLK,), lambda i: (i,)),
            pl.BlockSpec((1,), lambda i: (0,)),
        ],
        out_specs=pl.BlockSpec((G, 1), lambda i: (0, 0)),
        out_shape=jax.ShapeDtypeStruct((G, 1), jnp.float32),
    )(q0, q1, cnt, batch_pad, b2)


@functools.lru_cache(maxsize=1)
def _sc_kernels():
    mesh = plsc.VectorSubcoreMesh(
        core_axis_name="c", subcore_axis_name="s",
        num_cores=NC, num_subcores=NS)
    deg = pl.kernel(
        _deg_body,
        out_type=jax.ShapeDtypeStruct((NC, NPAD), jnp.float32),
        mesh=mesh,
        scratch_types=[
            pltpu.VMEM((CH, C), jnp.int32),
            pltpu.VMEM((C,), jnp.float32),
            pltpu.VMEM((RPT,), jnp.float32),
            pltpu.VMEM_SHARED((NPAD,), jnp.float32),
        ],
    )
    agg = pl.kernel(
        _agg_body,
        out_type=(jax.ShapeDtypeStruct((NPAD, D), jnp.float32),
                  jax.ShapeDtypeStruct((NPAD, D), jnp.float32)),
        mesh=mesh,
        scratch_types=[
            pltpu.VMEM((2, 2, C), jnp.int32),
            pltpu.VMEM((2, 2, C), jnp.int32),
            pltpu.VMEM((C, D), jnp.float32),
            pltpu.VMEM((C, D), jnp.float32),
            pltpu.VMEM_SHARED((NPAD, D), jnp.float32),
            pltpu.SemaphoreType.DMA,
            pltpu.SemaphoreType.DMA,
            pltpu.SemaphoreType.DMA,
        ],
    )
    agg1 = pl.kernel(
        _agg1_body,
        out_type=(jax.ShapeDtypeStruct((NPAD,), jnp.float32),
                  jax.ShapeDtypeStruct((NPAD,), jnp.float32)),
        mesh=mesh,
        scratch_types=[
            pltpu.VMEM((CH, C), jnp.int32),
            pltpu.VMEM((CH, C), jnp.int32),
            pltpu.VMEM((C,), jnp.float32),
            pltpu.VMEM((C,), jnp.float32),
            pltpu.VMEM_SHARED((NPAD,), jnp.float32),
            pltpu.VMEM_SHARED((NPAD,), jnp.float32),
            pltpu.SemaphoreType.DMA,
            pltpu.SemaphoreType.DMA,
        ],
    )
    return deg, agg, agg1


@jax.jit
def kernel(x, edge_index, batch, W1, b1, W2, b2):
    _deg, _agg, _agg1 = _sc_kernels()
    src = edge_index[0].astype(jnp.int32)
    dst = edge_index[1].astype(jnp.int32)
    loop = jnp.arange(NPAD, dtype=jnp.int32)
    pad_e = N + (jnp.arange(EPAD - E - NPAD, dtype=jnp.int32) % (NPAD - N))
    srcm = jnp.concatenate([src, loop, pad_e]).reshape(NW, CH, C)
    dstm = jnp.concatenate([dst, loop, pad_e]).reshape(NW, CH, C)
    x_pad = jnp.concatenate([x, jnp.zeros((NPAD - N, D), jnp.float32)], axis=0)
    batch_pad = jnp.concatenate(
        [batch.astype(jnp.int32), jnp.full((NPAD - N,), G, jnp.int32)])

    cnt = _deg(dstm)
    g1 = _dense1(x_pad, cnt, W1)
    p0, p1 = _agg(g1, srcm, dstm)
    g2 = _dense2(p0, p1, cnt, W2, b1)
    q0, q1 = _agg1(g2, srcm, dstm)
    return _dense3(q0, q1, cnt, batch_pad, b2)
